# Initial kernel scaffold; baseline (speedup 1.0000x reference)
import jax
import jax.numpy as jnp
from jax import lax
from jax.experimental import pallas as pl
from jax.experimental.pallas import tpu as pltpu
from jax.experimental.pallas import tpu_sc as plsc

N_NODES = 10000
N_EDGES = 160000
D = 256
BR = 1000
ROWS128 = N_EDGES // 128
CHUNK_ROWS = 5
NCHUNKS = ROWS128 // CHUNK_ROWS
NS = 16
RPT = N_NODES // NS

_sc_mesh = plsc.VectorSubcoreMesh(core_axis_name="c", subcore_axis_name="s")


def _make_sc_scatter(with_counts):
    out_type = [jax.ShapeDtypeStruct((2, N_NODES, 128), jnp.float32)]
    if with_counts:
        out_type.append(jax.ShapeDtypeStruct((N_NODES, 16), jnp.float32))

    scratch = [
        pltpu.VMEM((CHUNK_ROWS, 128), jnp.int32),
        pltpu.VMEM((CHUNK_ROWS, 128), jnp.int32),
        pltpu.VMEM((CHUNK_ROWS * 128, 128), jnp.float32),
        pltpu.VMEM_SHARED((N_NODES, 128), jnp.float32),
        pltpu.VMEM_SHARED((N_NODES, 16), jnp.float32),
        pltpu.SemaphoreType.DMA,
    ]

    def body(ytab, srcs, dst3, z128, z16, ones16, *rest):
        if with_counts:
            agg_out, cnt_out, srcv, dstv, rows, accum, cshr, sem = rest
        else:
            agg_out, srcv, dstv, rows, accum, cshr, sem = rest
        cid = lax.axis_index("c")
        sid = lax.axis_index("s")

        pltpu.sync_copy(z128.at[pl.ds(sid * RPT, RPT)],
                        accum.at[pl.ds(sid * RPT, RPT)])
        if with_counts:
            @pl.when(cid == 0)
            def _():
                pltpu.sync_copy(z16.at[pl.ds(sid * RPT, RPT)],
                                cshr.at[pl.ds(sid * RPT, RPT)])
        plsc.subcore_barrier()

        nit = (NCHUNKS + NS - 1 - sid) // NS

        def chunk(j, carry):
            ci = sid + j * NS
            c0 = ci * CHUNK_ROWS
            pltpu.sync_copy(srcs.at[cid, pl.ds(c0, CHUNK_ROWS)], srcv)
            pltpu.sync_copy(dst3.at[pl.ds(c0, CHUNK_ROWS)], dstv)
            descs = [
                pltpu.async_copy(ytab.at[srcv.at[k]],
                                 rows.at[pl.ds(k * 128, 128)], sem)
                for k in range(CHUNK_ROWS)
            ]
            for d_ in descs:
                d_.wait()
            for k in range(CHUNK_ROWS):
                pltpu.sync_copy(rows.at[pl.ds(k * 128, 128)],
                                accum.at[dstv.at[k]], add=True)
            if with_counts:
                @pl.when(cid == 0)
                def _():
                    for k in range(CHUNK_ROWS):
                        pltpu.sync_copy(ones16, cshr.at[dstv.at[k]],
                                        add=True)
            return carry

        lax.fori_loop(0, nit, chunk, 0)
        plsc.subcore_barrier()

        pltpu.sync_copy(accum.at[pl.ds(sid * RPT, RPT)],
                        agg_out.at[cid, pl.ds(sid * RPT, RPT)])
        if with_counts:
            @pl.when(cid == 0)
            def _():
                pltpu.sync_copy(cshr.at[pl.ds(sid * RPT, RPT)],
                                cnt_out.at[pl.ds(sid * RPT, RPT)])

    return pl.kernel(body, out_type=out_type, mesh=_sc_mesh,
                     scratch_types=scratch)


_sc_scatter_cnt = _make_sc_scatter(True)
_sc_scatter = _make_sc_scatter(False)


def _mm_split(x, wcat):
    def body(x_ref, w_ref, ytab_ref, skip_ref):
        acc = lax.dot_general(x_ref[...], w_ref[...],
                              (((1,), (0,)), ((), ())),
                              preferred_element_type=jnp.float32)
        ytab_ref[0] = acc[:, :128]
        ytab_ref[1] = acc[:, 128:256]
        skip_ref[...] = acc[:, 256:]

    return pl.pallas_call(
        body,
        grid=(N_NODES // BR,),
        in_specs=[pl.BlockSpec((BR, D), lambda i: (i, 0)),
                  pl.BlockSpec((D, 2 * D), lambda i: (0, 0))],
        out_specs=[pl.BlockSpec((2, BR, 128), lambda i: (0, i, 0)),
                   pl.BlockSpec((BR, D), lambda i: (i, 0))],
        out_shape=[jax.ShapeDtypeStruct((2, N_NODES, 128), jnp.float32),
                   jax.ShapeDtypeStruct((N_NODES, D), jnp.float32)],
    )(x, wcat)


def _fused_mid(agg, cnt, skip, bvec, wcat):
    def body(agg_ref, cnt_ref, skip_ref, b_ref, w_ref, ytab_ref, skip_o_ref):
        inv = 1.0 / jnp.maximum(cnt_ref[:, 0:1], 1.0)
        full = jnp.concatenate([agg_ref[0], agg_ref[1]], axis=1)
        x1 = jnp.maximum(full * inv + skip_ref[...] + b_ref[...], 0.0)
        acc = lax.dot_general(x1, w_ref[...],
                              (((1,), (0,)), ((), ())),
                              preferred_element_type=jnp.float32)
        ytab_ref[0] = acc[:, :128]
        ytab_ref[1] = acc[:, 128:256]
        skip_o_ref[...] = acc[:, 256:]

    return pl.pallas_call(
        body,
        grid=(N_NODES // BR,),
        in_specs=[pl.BlockSpec((2, BR, 128), lambda i: (0, i, 0)),
                  pl.BlockSpec((BR, 16), lambda i: (i, 0)),
                  pl.BlockSpec((BR, D), lambda i: (i, 0)),
                  pl.BlockSpec((1, D), lambda i: (0, 0)),
                  pl.BlockSpec((D, 2 * D), lambda i: (0, 0))],
        out_specs=[pl.BlockSpec((2, BR, 128), lambda i: (0, i, 0)),
                   pl.BlockSpec((BR, D), lambda i: (i, 0))],
        out_shape=[jax.ShapeDtypeStruct((2, N_NODES, 128), jnp.float32),
                   jax.ShapeDtypeStruct((N_NODES, D), jnp.float32)],
    )(agg, cnt, skip, bvec, wcat)


def _combine(agg0, cnt0, skip0, b0, agg1, cnt1, skip1, b1,
             me, pw1, pb1, pw2r):
    def body(a0_ref, c0_ref, s0_ref, b0_ref, a1_ref, c1_ref, s1_ref, b1_ref,
             me_ref, pw1_ref, pb1_ref, pw2_ref, out_ref):
        inv0 = 1.0 / jnp.maximum(c0_ref[:, 0:1], 1.0)
        full0 = jnp.concatenate([a0_ref[0], a0_ref[1]], axis=1)
        x0 = jnp.maximum(full0 * inv0 + s0_ref[...] + b0_ref[...], 0.0)
        inv1 = 1.0 / jnp.maximum(c1_ref[:, 0:1], 1.0)
        full1 = jnp.concatenate([a1_ref[0], a1_ref[1]], axis=1)
        x1 = jnp.maximum(full1 * inv1 + s1_ref[...] + b1_ref[...], 0.0)

        h = jnp.tanh(lax.dot_general(me_ref[...], pw1_ref[...],
                                     (((1,), (0,)), ((), ())),
                                     preferred_element_type=jnp.float32)
                     + pb1_ref[...])
        s = jnp.sum(h * pw2_ref[...], axis=1, keepdims=True)
        m = jnp.maximum(s[0:1], s[1:2])
        e0 = jnp.exp(s[0:1] - m)
        e1 = jnp.exp(s[1:2] - m)
        den = e0 + e1
        out_ref[...] = x0 * (e0 / den) + x1 * (e1 / den)

    return pl.pallas_call(
        body,
        grid=(N_NODES // BR,),
        in_specs=[pl.BlockSpec((2, BR, 128), lambda i: (0, i, 0)),
                  pl.BlockSpec((BR, 16), lambda i: (i, 0)),
                  pl.BlockSpec((BR, D), lambda i: (i, 0)),
                  pl.BlockSpec((1, D), lambda i: (0, 0)),
                  pl.BlockSpec((2, BR, 128), lambda i: (0, i, 0)),
                  pl.BlockSpec((BR, 16), lambda i: (i, 0)),
                  pl.BlockSpec((BR, D), lambda i: (i, 0)),
                  pl.BlockSpec((1, D), lambda i: (0, 0)),
                  pl.BlockSpec((2, 128), lambda i: (0, 0)),
                  pl.BlockSpec((128, D), lambda i: (0, 0)),
                  pl.BlockSpec((1, D), lambda i: (0, 0)),
                  pl.BlockSpec((1, D), lambda i: (0, 0))],
        out_specs=pl.BlockSpec((BR, D), lambda i: (i, 0)),
        out_shape=jax.ShapeDtypeStruct((N_NODES, D), jnp.float32),
    )(agg0, cnt0, skip0, b0, agg1, cnt1, skip1, b1, me, pw1, pb1, pw2r)


def kernel(E, metapath_emb, edge_index_0, eids_0, edge_index_1, eids_1,
           ifdropout, W00, R00, B00, W01, R01, B01, W10, R10, B10,
           W11, R11, B11, PW1, PB1, PW2):
    z128 = jnp.zeros((N_NODES, 128), jnp.float32)
    z16 = jnp.zeros((N_NODES, 16), jnp.float32)
    ones16 = jnp.ones((128, 16), jnp.float32)

    finals = []
    for ei, layers in ((edge_index_0, ((W00, R00, B00), (W01, R01, B01))),
                       (edge_index_1, ((W10, R10, B10), (W11, R11, B11)))):
        src = ei[0]
        dst = ei[1]
        srcs = jnp.stack([src, src + N_NODES]).reshape(2, ROWS128, 128)
        dst3 = dst.reshape(ROWS128, 128)

        (w0, r0, b0), (w1, r1, b1) = layers
        wcat1 = jnp.concatenate([w0[0], r0], axis=1)
        wcat2 = jnp.concatenate([w1[0], r1], axis=1)

        ytab1, skip1 = _mm_split(E, wcat1)
        agg1, cnt = _sc_scatter_cnt(ytab1.reshape(2 * N_NODES, 128),
                                    srcs, dst3, z128, z16, ones16)
        ytab2, skip2 = _fused_mid(agg1, cnt, skip1,
                                  b0.reshape(1, D), wcat2)
        agg2 = _sc_scatter(ytab2.reshape(2 * N_NODES, 128),
                           srcs, dst3, z128, z16, ones16)
        if isinstance(agg2, (list, tuple)):
            agg2 = agg2[0]
        finals.append((agg2, cnt, skip2, b1.reshape(1, D)))

    (a0, c0, s0, bb0), (a1, c1, s1, bb1) = finals
    return _combine(a0, c0, s0, bb0, a1, c1, s1, bb1,
                    metapath_emb, PW1, PB1.reshape(1, D),
                    PW2.reshape(1, D))


# speedup vs baseline: 2.3999x; 2.3999x over previous
import jax
import jax.numpy as jnp
from jax import lax
from jax.experimental import pallas as pl
from jax.experimental.pallas import tpu as pltpu
from jax.experimental.pallas import tpu_sc as plsc

N_NODES = 10000
N_EDGES = 160000
D = 256
BR = 1000
NS = 16
PAD_EDGES = 163840
ROWS128 = PAD_EDGES // 128
CHUNK_ROWS = 8
NCHUNKS = ROWS128 // CHUNK_ROWS
NIT = NCHUNKS // NS
SUB = 2
ACC_ROWS = N_NODES + 8
TRASH = N_NODES
S_STRIDE = 624
S_SIZE = 640

_sc_mesh = plsc.VectorSubcoreMesh(core_axis_name="c", subcore_axis_name="s")


def _sc_scatter_body(ytab, srcs, dst3, z128, agg_out,
                     srcv, dstv, rows, accum, sem):
    cid = lax.axis_index("c")
    sid = lax.axis_index("s")

    pltpu.sync_copy(z128.at[pl.ds(sid * S_STRIDE, S_SIZE)],
                    accum.at[pl.ds(sid * S_STRIDE, S_SIZE)])
    plsc.subcore_barrier()

    def chunk(j, carry):
        ci = sid * NIT + j
        c0 = ci * CHUNK_ROWS
        pltpu.sync_copy(srcs.at[cid, pl.ds(c0, CHUNK_ROWS)], srcv)
        pltpu.sync_copy(dst3.at[pl.ds(c0, CHUNK_ROWS)], dstv)
        for h in range(CHUNK_ROWS // SUB):
            descs = [
                pltpu.async_copy(ytab.at[srcv.at[h * SUB + k]],
                                 rows.at[pl.ds(k * 128, 128)], sem)
                for k in range(SUB)
            ]
            for d_ in descs:
                d_.wait()
            for k in range(SUB):
                pltpu.sync_copy(rows.at[pl.ds(k * 128, 128)],
                                accum.at[dstv.at[h * SUB + k]],
                                add=True)
        return carry

    lax.fori_loop(0, NIT, chunk, 0)
    plsc.subcore_barrier()

    pltpu.sync_copy(accum.at[pl.ds(sid * S_STRIDE, S_SIZE)],
                    agg_out.at[cid, pl.ds(sid * S_STRIDE, S_SIZE)])


_sc_scatter = pl.kernel(
    _sc_scatter_body,
    out_type=jax.ShapeDtypeStruct((2, N_NODES, 128), jnp.float32),
    mesh=_sc_mesh,
    scratch_types=[
        pltpu.VMEM((CHUNK_ROWS, 128), jnp.int32),
        pltpu.VMEM((CHUNK_ROWS, 128), jnp.int32),
        pltpu.VMEM((SUB * 128, 128), jnp.float32),
        pltpu.VMEM_SHARED((ACC_ROWS, 128), jnp.float32),
        pltpu.SemaphoreType.DMA,
    ])


def _sc_counts_body(dsts, z128, ones128, cnt_out, dstv, onev, cshr, sem):
    cid = lax.axis_index("c")
    sid = lax.axis_index("s")
    pltpu.sync_copy(ones128, onev)
    pltpu.sync_copy(z128.at[pl.ds(sid * S_STRIDE, S_SIZE)],
                    cshr.at[pl.ds(sid * S_STRIDE, S_SIZE)])
    plsc.subcore_barrier()

    def chunk(j, carry):
        c0 = (sid * NIT + j) * CHUNK_ROWS
        pltpu.sync_copy(dsts.at[cid, pl.ds(c0, CHUNK_ROWS)], dstv)
        for k in range(CHUNK_ROWS):
            pltpu.sync_copy(onev, cshr.at[dstv.at[k]], add=True)
        return carry

    lax.fori_loop(0, NIT, chunk, 0)
    plsc.subcore_barrier()

    pltpu.sync_copy(cshr.at[pl.ds(sid * S_STRIDE, S_SIZE)],
                    cnt_out.at[cid, pl.ds(sid * S_STRIDE, S_SIZE)])


_sc_counts = pl.kernel(
    _sc_counts_body,
    out_type=jax.ShapeDtypeStruct((2, N_NODES, 128), jnp.float32),
    mesh=_sc_mesh,
    scratch_types=[
        pltpu.VMEM((CHUNK_ROWS, 128), jnp.int32),
        pltpu.VMEM((128, 128), jnp.float32),
        pltpu.VMEM_SHARED((ACC_ROWS, 128), jnp.float32),
        pltpu.SemaphoreType.DMA,
    ])


def _mm_split(x, wcat):
    def body(x_ref, w_ref, ytab_ref, skip_ref):
        acc = lax.dot_general(x_ref[...], w_ref[...],
                              (((1,), (0,)), ((), ())),
                              precision=lax.Precision.HIGHEST,
                              preferred_element_type=jnp.float32)
        ytab_ref[0] = acc[:, :128]
        ytab_ref[1] = acc[:, 128:256]
        skip_ref[...] = acc[:, 256:]

    return pl.pallas_call(
        body,
        grid=(N_NODES // BR,),
        in_specs=[pl.BlockSpec((BR, D), lambda i: (i, 0)),
                  pl.BlockSpec((D, 2 * D), lambda i: (0, 0))],
        out_specs=[pl.BlockSpec((2, BR, 128), lambda i: (0, i, 0)),
                   pl.BlockSpec((BR, D), lambda i: (i, 0))],
        out_shape=[jax.ShapeDtypeStruct((2, N_NODES, 128), jnp.float32),
                   jax.ShapeDtypeStruct((N_NODES, D), jnp.float32)],
    )(x, wcat)


def _fused_mid(agg, cnt, skip, bvec, wcat):
    def body(agg_ref, cnt_ref, skip_ref, b_ref, w_ref, ytab_ref, skip_o_ref):
        inv = 1.0 / jnp.maximum(cnt_ref[:, 0:1], 1.0)
        full = jnp.concatenate([agg_ref[0], agg_ref[1]], axis=1)
        x1 = jnp.maximum(full * inv + skip_ref[...] + b_ref[...], 0.0)
        acc = lax.dot_general(x1, w_ref[...],
                              (((1,), (0,)), ((), ())),
                              precision=lax.Precision.HIGHEST,
                              preferred_element_type=jnp.float32)
        ytab_ref[0] = acc[:, :128]
        ytab_ref[1] = acc[:, 128:256]
        skip_o_ref[...] = acc[:, 256:]

    return pl.pallas_call(
        body,
        grid=(N_NODES // BR,),
        in_specs=[pl.BlockSpec((2, BR, 128), lambda i: (0, i, 0)),
                  pl.BlockSpec((BR, 128), lambda i: (i, 0)),
                  pl.BlockSpec((BR, D), lambda i: (i, 0)),
                  pl.BlockSpec((1, D), lambda i: (0, 0)),
                  pl.BlockSpec((D, 2 * D), lambda i: (0, 0))],
        out_specs=[pl.BlockSpec((2, BR, 128), lambda i: (0, i, 0)),
                   pl.BlockSpec((BR, D), lambda i: (i, 0))],
        out_shape=[jax.ShapeDtypeStruct((2, N_NODES, 128), jnp.float32),
                   jax.ShapeDtypeStruct((N_NODES, D), jnp.float32)],
    )(agg, cnt, skip, bvec, wcat)


def _combine(agg0, cnt0, skip0, b0, agg1, cnt1, skip1, b1,
             me, pw1, pb1, pw2r):
    def body(a0_ref, c0_ref, s0_ref, b0_ref, a1_ref, c1_ref, s1_ref, b1_ref,
             me_ref, pw1_ref, pb1_ref, pw2_ref, out_ref):
        inv0 = 1.0 / jnp.maximum(c0_ref[:, 0:1], 1.0)
        full0 = jnp.concatenate([a0_ref[0], a0_ref[1]], axis=1)
        x0 = jnp.maximum(full0 * inv0 + s0_ref[...] + b0_ref[...], 0.0)
        inv1 = 1.0 / jnp.maximum(c1_ref[:, 0:1], 1.0)
        full1 = jnp.concatenate([a1_ref[0], a1_ref[1]], axis=1)
        x1 = jnp.maximum(full1 * inv1 + s1_ref[...] + b1_ref[...], 0.0)

        h = jnp.tanh(lax.dot_general(me_ref[...], pw1_ref[...],
                                     (((1,), (0,)), ((), ())),
                                     precision=lax.Precision.HIGHEST,
                                     preferred_element_type=jnp.float32)
                     + pb1_ref[...])
        s = jnp.sum(h * pw2_ref[...], axis=1, keepdims=True)
        m = jnp.maximum(s[0:1], s[1:2])
        e0 = jnp.exp(s[0:1] - m)
        e1 = jnp.exp(s[1:2] - m)
        den = e0 + e1
        out_ref[...] = x0 * (e0 / den) + x1 * (e1 / den)

    return pl.pallas_call(
        body,
        grid=(N_NODES // BR,),
        in_specs=[pl.BlockSpec((2, BR, 128), lambda i: (0, i, 0)),
                  pl.BlockSpec((BR, 128), lambda i: (i, 0)),
                  pl.BlockSpec((BR, D), lambda i: (i, 0)),
                  pl.BlockSpec((1, D), lambda i: (0, 0)),
                  pl.BlockSpec((2, BR, 128), lambda i: (0, i, 0)),
                  pl.BlockSpec((BR, 128), lambda i: (i, 0)),
                  pl.BlockSpec((BR, D), lambda i: (i, 0)),
                  pl.BlockSpec((1, D), lambda i: (0, 0)),
                  pl.BlockSpec((2, 128), lambda i: (0, 0)),
                  pl.BlockSpec((128, D), lambda i: (0, 0)),
                  pl.BlockSpec((1, D), lambda i: (0, 0)),
                  pl.BlockSpec((1, D), lambda i: (0, 0))],
        out_specs=pl.BlockSpec((BR, D), lambda i: (i, 0)),
        out_shape=jax.ShapeDtypeStruct((N_NODES, D), jnp.float32),
    )(agg0, cnt0, skip0, b0, agg1, cnt1, skip1, b1, me, pw1, pb1, pw2r)


def kernel(E, metapath_emb, edge_index_0, eids_0, edge_index_1, eids_1,
           ifdropout, W00, R00, B00, W01, R01, B01, W10, R10, B10,
           W11, R11, B11, PW1, PB1, PW2):
    z128 = jnp.zeros((N_NODES, 128), jnp.float32)
    ones128 = jnp.ones((128, 128), jnp.float32)
    pad = PAD_EDGES - N_EDGES

    srcs_l, dst3_l = [], []
    for ei in (edge_index_0, edge_index_1):
        src_p = jnp.concatenate([ei[0], jnp.zeros((pad,), jnp.int32)])
        dst_p = jnp.concatenate([ei[1], jnp.full((pad,), TRASH, jnp.int32)])
        srcs_l.append(jnp.stack([src_p, src_p + N_NODES])
                      .reshape(2, ROWS128, 128))
        dst3_l.append(dst_p.reshape(ROWS128, 128))

    cnts = _sc_counts(jnp.stack(dst3_l), z128, ones128)

    finals = []
    for m, layers in ((0, ((W00, R00, B00), (W01, R01, B01))),
                      (1, ((W10, R10, B10), (W11, R11, B11)))):
        srcs, dst3, cnt = srcs_l[m], dst3_l[m], cnts[m]
        (w0, r0, b0), (w1, r1, b1) = layers
        wcat1 = jnp.concatenate([w0[0], r0], axis=1)
        wcat2 = jnp.concatenate([w1[0], r1], axis=1)

        ytab1, skip1 = _mm_split(E, wcat1)
        agg1 = _sc_scatter(ytab1.reshape(2 * N_NODES, 128),
                           srcs, dst3, z128)
        ytab2, skip2 = _fused_mid(agg1, cnt, skip1,
                                  b0.reshape(1, D), wcat2)
        agg2 = _sc_scatter(ytab2.reshape(2 * N_NODES, 128),
                           srcs, dst3, z128)
        finals.append((agg2, cnt, skip2, b1.reshape(1, D)))

    (a0, c0, s0, bb0), (a1, c1, s1, bb1) = finals
    return _combine(a0, c0, s0, bb0, a1, c1, s1, bb1,
                    metapath_emb, PW1, PB1.reshape(1, D),
                    PW2.reshape(1, D))


# speedup vs baseline: 2.6213x; 1.0923x over previous
import jax
import jax.numpy as jnp
from jax import lax
from jax.experimental import pallas as pl
from jax.experimental.pallas import tpu as pltpu
from jax.experimental.pallas import tpu_sc as plsc

N_NODES = 10000
N_EDGES = 160000
D = 256
BR = 1000
NS = 16
PAD_EDGES = 163840
ROWS128 = PAD_EDGES // 128
CHUNK_ROWS = 8
NCHUNKS = ROWS128 // CHUNK_ROWS
NIT = NCHUNKS // NS
SUB = 2
ACC_ROWS = N_NODES + 8
TRASH = N_NODES
S_STRIDE = 624
S_SIZE = 640

_sc_mesh = plsc.VectorSubcoreMesh(core_axis_name="c", subcore_axis_name="s")


def _sc_scatter_body(ytab, srcs, dst3, z128, agg_out,
                     srcv, dstv, rows, accum, sem0, sem1):
    cid = lax.axis_index("c")
    sid = lax.axis_index("s")

    pltpu.sync_copy(z128.at[pl.ds(sid * S_STRIDE, S_SIZE)],
                    accum.at[pl.ds(sid * S_STRIDE, S_SIZE)])
    plsc.subcore_barrier()

    sems = (sem0, sem1)

    def gather(k, buf):
        return pltpu.async_copy(ytab.at[srcv.at[k]],
                                rows.at[pl.ds(buf * 128, 128)],
                                sems[buf])

    def scatter(k, buf):
        pltpu.sync_copy(rows.at[pl.ds(buf * 128, 128)],
                        accum.at[dstv.at[k]], add=True)

    def chunk(j, carry):
        c0 = (sid * NIT + j) * CHUNK_ROWS
        pltpu.sync_copy(srcs.at[cid, pl.ds(c0, CHUNK_ROWS)], srcv)
        pltpu.sync_copy(dst3.at[pl.ds(c0, CHUNK_ROWS)], dstv)
        d = [gather(0, 0), None]
        for k in range(CHUNK_ROWS):
            buf = k % 2
            if k + 1 < CHUNK_ROWS:
                d[1 - buf] = gather(k + 1, 1 - buf)
            d[buf].wait()
            scatter(k, buf)
        return carry

    lax.fori_loop(0, NIT, chunk, 0)
    plsc.subcore_barrier()

    pltpu.sync_copy(accum.at[pl.ds(sid * S_STRIDE, S_SIZE)],
                    agg_out.at[cid, pl.ds(sid * S_STRIDE, S_SIZE)])


_sc_scatter = pl.kernel(
    _sc_scatter_body,
    out_type=jax.ShapeDtypeStruct((2, N_NODES, 128), jnp.float32),
    mesh=_sc_mesh,
    scratch_types=[
        pltpu.VMEM((CHUNK_ROWS, 128), jnp.int32),
        pltpu.VMEM((CHUNK_ROWS, 128), jnp.int32),
        pltpu.VMEM((2 * 128, 128), jnp.float32),
        pltpu.VMEM_SHARED((ACC_ROWS, 128), jnp.float32),
        pltpu.SemaphoreType.DMA,
        pltpu.SemaphoreType.DMA,
    ])


def _sc_counts_body(dsts, z128, ones128, cnt_out, dstv, onev, cshr, sem):
    cid = lax.axis_index("c")
    sid = lax.axis_index("s")
    pltpu.sync_copy(ones128, onev)
    pltpu.sync_copy(z128.at[pl.ds(sid * S_STRIDE, S_SIZE)],
                    cshr.at[pl.ds(sid * S_STRIDE, S_SIZE)])
    plsc.subcore_barrier()

    def chunk(j, carry):
        c0 = (sid * NIT + j) * CHUNK_ROWS
        pltpu.sync_copy(dsts.at[cid, pl.ds(c0, CHUNK_ROWS)], dstv)
        for k in range(CHUNK_ROWS):
            pltpu.sync_copy(onev, cshr.at[dstv.at[k]], add=True)
        return carry

    lax.fori_loop(0, NIT, chunk, 0)
    plsc.subcore_barrier()

    pltpu.sync_copy(cshr.at[pl.ds(sid * S_STRIDE, S_SIZE)],
                    cnt_out.at[cid, pl.ds(sid * S_STRIDE, S_SIZE)])


_sc_counts = pl.kernel(
    _sc_counts_body,
    out_type=jax.ShapeDtypeStruct((2, N_NODES, 128), jnp.float32),
    mesh=_sc_mesh,
    scratch_types=[
        pltpu.VMEM((CHUNK_ROWS, 128), jnp.int32),
        pltpu.VMEM((128, 128), jnp.float32),
        pltpu.VMEM_SHARED((ACC_ROWS, 128), jnp.float32),
        pltpu.SemaphoreType.DMA,
    ])


def _mm_split(x, wcat):
    def body(x_ref, w_ref, ytab_ref, skip_ref):
        acc = lax.dot_general(x_ref[...], w_ref[...],
                              (((1,), (0,)), ((), ())),
                              precision=lax.Precision.HIGHEST,
                              preferred_element_type=jnp.float32)
        ytab_ref[0] = acc[:, :128]
        ytab_ref[1] = acc[:, 128:256]
        skip_ref[...] = acc[:, 256:]

    return pl.pallas_call(
        body,
        grid=(N_NODES // BR,),
        in_specs=[pl.BlockSpec((BR, D), lambda i: (i, 0)),
                  pl.BlockSpec((D, 2 * D), lambda i: (0, 0))],
        out_specs=[pl.BlockSpec((2, BR, 128), lambda i: (0, i, 0)),
                   pl.BlockSpec((BR, D), lambda i: (i, 0))],
        out_shape=[jax.ShapeDtypeStruct((2, N_NODES, 128), jnp.float32),
                   jax.ShapeDtypeStruct((N_NODES, D), jnp.float32)],
    )(x, wcat)


def _fused_mid(agg, cnt, skip, bvec, wcat):
    def body(agg_ref, cnt_ref, skip_ref, b_ref, w_ref, ytab_ref, skip_o_ref):
        inv = 1.0 / jnp.maximum(cnt_ref[:, 0:1], 1.0)
        full = jnp.concatenate([agg_ref[0], agg_ref[1]], axis=1)
        x1 = jnp.maximum(full * inv + skip_ref[...] + b_ref[...], 0.0)
        acc = lax.dot_general(x1, w_ref[...],
                              (((1,), (0,)), ((), ())),
                              precision=lax.Precision.HIGHEST,
                              preferred_element_type=jnp.float32)
        ytab_ref[0] = acc[:, :128]
        ytab_ref[1] = acc[:, 128:256]
        skip_o_ref[...] = acc[:, 256:]

    return pl.pallas_call(
        body,
        grid=(N_NODES // BR,),
        in_specs=[pl.BlockSpec((2, BR, 128), lambda i: (0, i, 0)),
                  pl.BlockSpec((BR, 128), lambda i: (i, 0)),
                  pl.BlockSpec((BR, D), lambda i: (i, 0)),
                  pl.BlockSpec((1, D), lambda i: (0, 0)),
                  pl.BlockSpec((D, 2 * D), lambda i: (0, 0))],
        out_specs=[pl.BlockSpec((2, BR, 128), lambda i: (0, i, 0)),
                   pl.BlockSpec((BR, D), lambda i: (i, 0))],
        out_shape=[jax.ShapeDtypeStruct((2, N_NODES, 128), jnp.float32),
                   jax.ShapeDtypeStruct((N_NODES, D), jnp.float32)],
    )(agg, cnt, skip, bvec, wcat)


def _combine(agg0, cnt0, skip0, b0, agg1, cnt1, skip1, b1,
             me, pw1, pb1, pw2r):
    def body(a0_ref, c0_ref, s0_ref, b0_ref, a1_ref, c1_ref, s1_ref, b1_ref,
             me_ref, pw1_ref, pb1_ref, pw2_ref, out_ref):
        inv0 = 1.0 / jnp.maximum(c0_ref[:, 0:1], 1.0)
        full0 = jnp.concatenate([a0_ref[0], a0_ref[1]], axis=1)
        x0 = jnp.maximum(full0 * inv0 + s0_ref[...] + b0_ref[...], 0.0)
        inv1 = 1.0 / jnp.maximum(c1_ref[:, 0:1], 1.0)
        full1 = jnp.concatenate([a1_ref[0], a1_ref[1]], axis=1)
        x1 = jnp.maximum(full1 * inv1 + s1_ref[...] + b1_ref[...], 0.0)

        h = jnp.tanh(lax.dot_general(me_ref[...], pw1_ref[...],
                                     (((1,), (0,)), ((), ())),
                                     precision=lax.Precision.HIGHEST,
                                     preferred_element_type=jnp.float32)
                     + pb1_ref[...])
        s = jnp.sum(h * pw2_ref[...], axis=1, keepdims=True)
        m = jnp.maximum(s[0:1], s[1:2])
        e0 = jnp.exp(s[0:1] - m)
        e1 = jnp.exp(s[1:2] - m)
        den = e0 + e1
        out_ref[...] = x0 * (e0 / den) + x1 * (e1 / den)

    return pl.pallas_call(
        body,
        grid=(N_NODES // BR,),
        in_specs=[pl.BlockSpec((2, BR, 128), lambda i: (0, i, 0)),
                  pl.BlockSpec((BR, 128), lambda i: (i, 0)),
                  pl.BlockSpec((BR, D), lambda i: (i, 0)),
                  pl.BlockSpec((1, D), lambda i: (0, 0)),
                  pl.BlockSpec((2, BR, 128), lambda i: (0, i, 0)),
                  pl.BlockSpec((BR, 128), lambda i: (i, 0)),
                  pl.BlockSpec((BR, D), lambda i: (i, 0)),
                  pl.BlockSpec((1, D), lambda i: (0, 0)),
                  pl.BlockSpec((2, 128), lambda i: (0, 0)),
                  pl.BlockSpec((128, D), lambda i: (0, 0)),
                  pl.BlockSpec((1, D), lambda i: (0, 0)),
                  pl.BlockSpec((1, D), lambda i: (0, 0))],
        out_specs=pl.BlockSpec((BR, D), lambda i: (i, 0)),
        out_shape=jax.ShapeDtypeStruct((N_NODES, D), jnp.float32),
    )(agg0, cnt0, skip0, b0, agg1, cnt1, skip1, b1, me, pw1, pb1, pw2r)


def kernel(E, metapath_emb, edge_index_0, eids_0, edge_index_1, eids_1,
           ifdropout, W00, R00, B00, W01, R01, B01, W10, R10, B10,
           W11, R11, B11, PW1, PB1, PW2):
    z128 = jnp.zeros((N_NODES, 128), jnp.float32)
    ones128 = jnp.ones((128, 128), jnp.float32)
    pad = PAD_EDGES - N_EDGES

    srcs_l, dst3_l = [], []
    for ei in (edge_index_0, edge_index_1):
        src_p = jnp.concatenate([ei[0], jnp.zeros((pad,), jnp.int32)])
        dst_p = jnp.concatenate([ei[1], jnp.full((pad,), TRASH, jnp.int32)])
        srcs_l.append(jnp.stack([src_p, src_p + N_NODES])
                      .reshape(2, ROWS128, 128))
        dst3_l.append(dst_p.reshape(ROWS128, 128))

    cnts = _sc_counts(jnp.stack(dst3_l), z128, ones128)

    finals = []
    for m, layers in ((0, ((W00, R00, B00), (W01, R01, B01))),
                      (1, ((W10, R10, B10), (W11, R11, B11)))):
        srcs, dst3, cnt = srcs_l[m], dst3_l[m], cnts[m]
        (w0, r0, b0), (w1, r1, b1) = layers
        wcat1 = jnp.concatenate([w0[0], r0], axis=1)
        wcat2 = jnp.concatenate([w1[0], r1], axis=1)

        ytab1, skip1 = _mm_split(E, wcat1)
        agg1 = _sc_scatter(ytab1.reshape(2 * N_NODES, 128),
                           srcs, dst3, z128)
        ytab2, skip2 = _fused_mid(agg1, cnt, skip1,
                                  b0.reshape(1, D), wcat2)
        agg2 = _sc_scatter(ytab2.reshape(2 * N_NODES, 128),
                           srcs, dst3, z128)
        finals.append((agg2, cnt, skip2, b1.reshape(1, D)))

    (a0, c0, s0, bb0), (a1, c1, s1, bb1) = finals
    return _combine(a0, c0, s0, bb0, a1, c1, s1, bb1,
                    metapath_emb, PW1, PB1.reshape(1, D),
                    PW2.reshape(1, D))
